# baseline (speedup 1.0000x reference)
import functools

import jax
import jax.numpy as jnp
from jax import lax
from jax.experimental import pallas as pl
from jax.experimental.pallas import tpu as pltpu
from jax.experimental.pallas import tpu_sc as plsc

SIZE = 1000000
DIM = 32
B = 4096
L = 52
NLANES = 16

NC = 2
NS = 16
NW = NC * NS
BPW = B // NW
C = 16
NCHUNK = BPW // C
JBASES = (0, 16, 32, 35)

_mesh = plsc.VectorSubcoreMesh(
    core_axis_name="c", subcore_axis_name="s", num_cores=NC, num_subcores=NS
)


@functools.partial(
    pl.kernel,
    out_type=jax.ShapeDtypeStruct((B, L - 1), jnp.float32),
    mesh=_mesh,
    scratch_types=[
        pltpu.VMEM((BPW, L), jnp.int32),
        pltpu.VMEM((C * L, DIM), jnp.float32),
        pltpu.VMEM((C * L, DIM), jnp.float32),
        pltpu.VMEM((C, L - 1), jnp.float32),
        pltpu.SemaphoreType.DMA,
        pltpu.SemaphoreType.DMA,
    ],
    compiler_params=pltpu.CompilerParams(
        needs_layout_passes=False, use_tc_tiling_on_sc=False
    ),
)
def _sc_embed_dist(
    inputs_hbm, table_hbm, out_hbm, idx_all, rows_a, rows_b, out_v, sem_a, sem_b
):
    wid = lax.axis_index("s") * NC + lax.axis_index("c")
    base = wid * BPW
    pltpu.sync_copy(inputs_hbm.at[pl.ds(base, BPW)], idx_all)

    iota = lax.iota(jnp.int32, NLANES)

    def fire(k, rows_ref, sem):
        return [
            pltpu.async_copy(
                table_hbm.at[idx_all.at[k * C + r]],
                rows_ref.at[pl.ds(r * L, L)],
                sem,
            )
            for r in range(C)
        ]

    def compute(k, rows_ref):
        def row_body(r, carry):
            roff = r * L
            ridx = [roff + 1 + jb + iota for jb in JBASES]
            accs = [jnp.zeros((NLANES,), jnp.float32) for _ in JBASES]
            s_halves = (
                rows_ref[roff, pl.ds(0, NLANES)],
                rows_ref[roff, pl.ds(NLANES, NLANES)],
            )
            for d in range(DIM):
                sb = lax.broadcast(s_halves[d // NLANES][d % NLANES], (NLANES,))
                colv = jnp.full((NLANES,), d, jnp.int32)
                for g in range(len(JBASES)):
                    v = plsc.load_gather(rows_ref, [ridx[g], colv])
                    diff = v - sb
                    accs[g] = accs[g] + diff * diff
            for g, jb in enumerate(JBASES):
                out_v[r, pl.ds(jb, NLANES)] = -accs[g]
            return carry

        lax.fori_loop(0, C, row_body, 0)
        pltpu.sync_copy(out_v, out_hbm.at[pl.ds(base + k * C, C)])

    bufs = ((rows_a, sem_a), (rows_b, sem_b))
    pend = fire(0, *bufs[0])
    for k in range(NCHUNK):
        nxt = fire(k + 1, *bufs[(k + 1) % 2]) if k + 1 < NCHUNK else None
        for h in pend:
            h.wait()
        compute(k, bufs[k % 2][0])
        pend = nxt


def kernel(inputs, table):
    return _sc_embed_dist(inputs, table)
